# Initial kernel scaffold; baseline (speedup 1.0000x reference)
import jax
import jax.numpy as jnp
from jax.experimental import pallas as pl


def kernel(nodes, edges, node_table, edge_table):
    raise NotImplementedError("write your pallas kernel here")


# speedup vs baseline: 2.2614x; 2.2614x over previous
import functools

import jax
import jax.numpy as jnp
from jax import lax
from jax.experimental import pallas as pl
from jax.experimental.pallas import tpu as pltpu
from jax.experimental.pallas import tpu_sc as plsc

NODE_DIM = 64
EDGE_DIM = 32
NC = 2
NS = 16
NW = NC * NS
CHUNK = 512


def _emb_body(nodes_hbm, edges_hbm, node_tab, edge_tab, out_n, out_e,
              idx_n, idx_e, rows_n, rows_e, sem_n, sem_e, b_per_w):
    wid = lax.axis_index("s") * NC + lax.axis_index("c")
    base = wid * b_per_w
    steps = b_per_w // CHUNK

    def node_step(i, carry):
        off = base + i * CHUNK
        pltpu.sync_copy(nodes_hbm.at[pl.ds(off, CHUNK)], idx_n)
        pltpu.async_copy(node_tab.at[idx_n], rows_n, sem_n).wait()
        pltpu.sync_copy(rows_n, out_n.at[pl.ds(off, CHUNK)])
        return carry

    lax.fori_loop(0, steps, node_step, 0)

    def edge_step(i, carry):
        off = base + i * CHUNK
        pltpu.sync_copy(edges_hbm.at[pl.ds(off, CHUNK)], idx_e)
        pltpu.async_copy(edge_tab.at[idx_e], rows_e, sem_e).wait()
        pltpu.sync_copy(rows_e, out_e.at[pl.ds(off, CHUNK)])
        return carry

    lax.fori_loop(0, steps, edge_step, 0)


@jax.jit
def _embedding_lookup(nodes_flat, edges_flat, node_table, edge_table):
    b = nodes_flat.shape[0]
    b_per_w = b // NW
    mesh = plsc.VectorSubcoreMesh(core_axis_name="c", subcore_axis_name="s")
    run = pl.kernel(
        functools.partial(_emb_body, b_per_w=b_per_w),
        out_type=(
            jax.ShapeDtypeStruct((b, NODE_DIM), jnp.float32),
            jax.ShapeDtypeStruct((b, EDGE_DIM), jnp.float32),
        ),
        mesh=mesh,
        scratch_types=[
            pltpu.VMEM((CHUNK,), jnp.int32),
            pltpu.VMEM((CHUNK,), jnp.int32),
            pltpu.VMEM((CHUNK, NODE_DIM), jnp.float32),
            pltpu.VMEM((CHUNK, EDGE_DIM), jnp.float32),
            pltpu.SemaphoreType.DMA,
            pltpu.SemaphoreType.DMA,
        ],
        compiler_params=pltpu.CompilerParams(use_tc_tiling_on_sc=False),
    )
    return run(nodes_flat, edges_flat, node_table, edge_table)


def kernel(nodes, edges, node_table, edge_table):
    out_n, out_e = _embedding_lookup(
        nodes.reshape(-1), edges.reshape(-1), node_table, edge_table)
    return (out_n.reshape(nodes.shape + (NODE_DIM,)),
            out_e.reshape(edges.shape + (EDGE_DIM,)))


# speedup vs baseline: 2.3505x; 1.0394x over previous
import functools

import jax
import jax.numpy as jnp
from jax import lax
from jax.experimental import pallas as pl
from jax.experimental.pallas import tpu as pltpu
from jax.experimental.pallas import tpu_sc as plsc

NODE_DIM = 64
EDGE_DIM = 32
NC = 2
NS = 16
NW = NC * NS
CHUNK = 640


def _pipelined_gather(idx_hbm, tab_hbm, out_hbm, idx_v, rows_v, gsem, ssem,
                      base, steps):
    def fire(c, b):
        pltpu.sync_copy(idx_hbm.at[pl.ds(base + c * CHUNK, CHUNK)],
                        idx_v.at[b])
        pltpu.async_copy(tab_hbm.at[idx_v.at[b]], rows_v.at[b], gsem.at[b])

    def drain_gather(b):
        pltpu.make_async_copy(tab_hbm.at[pl.ds(0, CHUNK)], rows_v.at[b],
                              gsem.at[b]).wait()

    def store(c, b):
        pltpu.async_copy(rows_v.at[b], out_hbm.at[pl.ds(base + c * CHUNK,
                                                        CHUNK)], ssem.at[b])

    def drain_store(b):
        pltpu.make_async_copy(rows_v.at[b], out_hbm.at[pl.ds(base, CHUNK)],
                              ssem.at[b]).wait()

    npair = steps // 2
    fire(0, 0)

    def body(j, carry):
        @pl.when(j >= 1)
        def _():
            drain_store(1)
        fire(2 * j + 1, 1)
        drain_gather(0)
        store(2 * j, 0)

        @pl.when(j + 1 < npair)
        def _():
            drain_store(0)
            fire(2 * j + 2, 0)
        drain_gather(1)
        store(2 * j + 1, 1)
        return carry

    lax.fori_loop(0, npair, body, 0)
    drain_store(0)
    drain_store(1)


def _emb_body(nodes_hbm, edges_hbm, node_tab, edge_tab, out_n, out_e,
              idx_v, rows_n, rows_e, gsem, ssem, b_per_w):
    wid = lax.axis_index("s") * NC + lax.axis_index("c")
    base = wid * b_per_w
    steps = b_per_w // CHUNK
    _pipelined_gather(nodes_hbm, node_tab, out_n, idx_v, rows_n, gsem, ssem,
                      base, steps)
    _pipelined_gather(edges_hbm, edge_tab, out_e, idx_v, rows_e, gsem, ssem,
                      base, steps)


@jax.jit
def _embedding_lookup(nodes_flat, edges_flat, node_table, edge_table):
    b = nodes_flat.shape[0]
    b_per_w = b // NW
    mesh = plsc.VectorSubcoreMesh(core_axis_name="c", subcore_axis_name="s")
    run = pl.kernel(
        functools.partial(_emb_body, b_per_w=b_per_w),
        out_type=(
            jax.ShapeDtypeStruct((b, NODE_DIM), jnp.float32),
            jax.ShapeDtypeStruct((b, EDGE_DIM), jnp.float32),
        ),
        mesh=mesh,
        scratch_types=[
            pltpu.VMEM((2, CHUNK), jnp.int32),
            pltpu.VMEM((2, CHUNK, NODE_DIM), jnp.float32),
            pltpu.VMEM((2, CHUNK, EDGE_DIM), jnp.float32),
            pltpu.SemaphoreType.DMA((2,)),
            pltpu.SemaphoreType.DMA((2,)),
        ],
        compiler_params=pltpu.CompilerParams(use_tc_tiling_on_sc=False),
    )
    return run(nodes_flat, edges_flat, node_table, edge_table)


def kernel(nodes, edges, node_table, edge_table):
    out_n, out_e = _embedding_lookup(
        nodes.reshape(-1), edges.reshape(-1), node_table, edge_table)
    return (out_n.reshape(nodes.shape + (NODE_DIM,)),
            out_e.reshape(edges.shape + (EDGE_DIM,)))


# speedup vs baseline: 2.3765x; 1.0110x over previous
import functools

import jax
import jax.numpy as jnp
from jax import lax
from jax.experimental import pallas as pl
from jax.experimental.pallas import tpu as pltpu
from jax.experimental.pallas import tpu_sc as plsc

NODE_DIM = 64
EDGE_DIM = 32
NC = 2
NS = 16
NW = NC * NS
CH_N = 800
CH_E = 1600


def _phase(idx2d_hbm, tab_hbm, out_hbm, gsem, ssem, wid, steps, ch, dim):
    def scoped(idx_all, rows):
        pltpu.sync_copy(idx2d_hbm.at[pl.ds(wid * steps, steps)], idx_all)
        base = wid * steps * ch

        def fire(c, b):
            pltpu.async_copy(tab_hbm.at[idx_all.at[c]], rows.at[b],
                             gsem.at[b])

        def drain_gather(b):
            pltpu.make_async_copy(tab_hbm.at[pl.ds(0, ch)], rows.at[b],
                                  gsem.at[b]).wait()

        def store(c, b):
            pltpu.async_copy(rows.at[b],
                             out_hbm.at[pl.ds(base + c * ch, ch)],
                             ssem.at[b])

        def drain_store(b):
            pltpu.make_async_copy(rows.at[b], out_hbm.at[pl.ds(base, ch)],
                                  ssem.at[b]).wait()

        fire(0, 0)

        def body(j, carry):
            @pl.when(j >= 1)
            def _():
                drain_store(1)
            fire(2 * j + 1, 1)
            drain_gather(0)
            store(2 * j, 0)

            @pl.when(j + 1 < steps // 2)
            def _():
                drain_store(0)
                fire(2 * j + 2, 0)
            drain_gather(1)
            store(2 * j + 1, 1)
            return carry

        lax.fori_loop(0, steps // 2, body, 0)
        drain_store(0)
        drain_store(1)

    pl.run_scoped(
        scoped,
        idx_all=pltpu.VMEM((steps, ch), jnp.int32),
        rows=pltpu.VMEM((2, ch, dim), jnp.float32),
    )


def _emb_body(nodes_hbm, edges_hbm, node_tab, edge_tab, out_n, out_e,
              gsem, ssem, steps_n, steps_e):
    wid = lax.axis_index("s") * NC + lax.axis_index("c")
    _phase(nodes_hbm, node_tab, out_n, gsem, ssem, wid, steps_n, CH_N,
           NODE_DIM)
    _phase(edges_hbm, edge_tab, out_e, gsem, ssem, wid, steps_e, CH_E,
           EDGE_DIM)


@jax.jit
def _embedding_lookup(nodes_2d, edges_2d, node_table, edge_table):
    b = nodes_2d.shape[0] * nodes_2d.shape[1]
    steps_n = b // (NW * CH_N)
    steps_e = b // (NW * CH_E)
    mesh = plsc.VectorSubcoreMesh(core_axis_name="c", subcore_axis_name="s")
    run = pl.kernel(
        functools.partial(_emb_body, steps_n=steps_n, steps_e=steps_e),
        out_type=(
            jax.ShapeDtypeStruct((b, NODE_DIM), jnp.float32),
            jax.ShapeDtypeStruct((b, EDGE_DIM), jnp.float32),
        ),
        mesh=mesh,
        scratch_types=[
            pltpu.SemaphoreType.DMA((2,)),
            pltpu.SemaphoreType.DMA((2,)),
        ],
        compiler_params=pltpu.CompilerParams(use_tc_tiling_on_sc=False),
    )
    return run(nodes_2d, edges_2d, node_table, edge_table)


def kernel(nodes, edges, node_table, edge_table):
    out_n, out_e = _embedding_lookup(
        nodes.reshape(-1, CH_N), edges.reshape(-1, CH_E),
        node_table, edge_table)
    return (out_n.reshape(nodes.shape + (NODE_DIM,)),
            out_e.reshape(edges.shape + (EDGE_DIM,)))
